# Initial kernel scaffold; baseline (speedup 1.0000x reference)
import jax
import jax.numpy as jnp
from jax.experimental import pallas as pl


def kernel(feats, num_patches, patch_ids):
    raise NotImplementedError("write your pallas kernel here")


# speedup vs baseline: 1.8264x; 1.8264x over previous
import functools

import jax
import jax.numpy as jnp
from jax import lax
from jax.experimental import pallas as pl
from jax.experimental.pallas import tpu as pltpu
from jax.experimental.pallas import tpu_sc as plsc

B = 4
C = 192
HW = 384 * 384
CHW = C * HW
NUM_P = 256
ROWS = B * NUM_P
ELEMS = ROWS * C

_info = plsc.get_sparse_core_info()
NC, NS, L = _info.num_cores, _info.num_subcores, _info.num_lanes
NW = NC * NS
ROWS_PER_W = ROWS // NW
ELEMS_PER_W = ROWS_PER_W * C
IDX_MINOR = 128
IDX_MAJOR = ELEMS_PER_W // IDX_MINOR
OUT_ROWS = ELEMS // IDX_MINOR


@functools.partial(
    pl.kernel,
    out_type=jax.ShapeDtypeStruct((OUT_ROWS, IDX_MINOR), jnp.float32),
    mesh=plsc.VectorSubcoreMesh(core_axis_name="c", subcore_axis_name="s"),
    scratch_types=[
        pltpu.VMEM((ROWS_PER_W,), jnp.int32),
        pltpu.VMEM((IDX_MAJOR, IDX_MINOR), jnp.int32),
        pltpu.VMEM((IDX_MAJOR, IDX_MINOR), jnp.float32),
        pltpu.SemaphoreType.DMA,
    ],
)
def _sc_gather(feats_hbm, pids_hbm, out_hbm, pid_v, idx_v, rows_v, sem):
    wid = lax.axis_index("s") * NC + lax.axis_index("c")
    base_row = wid * ROWS_PER_W
    b = lax.shift_right_logical(wid, 3)
    base_b = b * CHW

    pltpu.sync_copy(pids_hbm.at[pl.ds(base_row, ROWS_PER_W)], pid_v)

    iota16 = lax.iota(jnp.int32, L)
    c_offs = [(iota16 + 16 * cc) * HW + base_b for cc in range(C // L)]

    for g in range(ROWS_PER_W // L):
        pid_vec = pid_v[pl.ds(g * L, L)]
        for r in range(L):
            pid = pid_vec[r]
            row = g * L + r
            for cc in range(C // L):
                u = (C // L) * row + cc
                j, k = divmod(u, IDX_MINOR // L)
                idx_v[j, pl.ds(k * L, L)] = c_offs[cc] + pid

    def fire(j, carry):
        pltpu.async_copy(feats_hbm.at[idx_v.at[j]], rows_v.at[j], sem)
        return carry

    lax.fori_loop(0, IDX_MAJOR, fire, 0)
    pltpu.make_async_copy(
        out_hbm.at[pl.ds(wid * IDX_MAJOR, IDX_MAJOR)], rows_v, sem
    ).wait()
    pltpu.sync_copy(rows_v, out_hbm.at[pl.ds(wid * IDX_MAJOR, IDX_MAJOR)])


def _norm_body(x_ref, o_ref):
    x = x_ref[...]
    s = jnp.sum(x * x, axis=1, keepdims=True)
    o_ref[...] = x / (jnp.sqrt(s) + 1e-7)


def _normalize(x):
    return pl.pallas_call(
        _norm_body,
        out_shape=jax.ShapeDtypeStruct((ROWS, C), jnp.float32),
    )(x)


def kernel(feats, num_patches, patch_ids):
    del num_patches
    feats_flat = feats.reshape(-1)
    pids_flat = patch_ids.reshape(-1)
    gathered = _sc_gather(feats_flat, pids_flat)
    normed = _normalize(gathered.reshape(ROWS, C))
    return normed.reshape(B, NUM_P, C), patch_ids


# speedup vs baseline: 22.3912x; 12.2600x over previous
import functools

import jax
import jax.numpy as jnp
from jax import lax
from jax.experimental import pallas as pl
from jax.experimental.pallas import tpu as pltpu
from jax.experimental.pallas import tpu_sc as plsc

B = 4
C = 192
HW = 384 * 384
CHW = C * HW
NUM_P = 256
ROWS = B * NUM_P
ELEMS = ROWS * C

_info = plsc.get_sparse_core_info()
NC, NS, L = _info.num_cores, _info.num_subcores, _info.num_lanes
NW = NC * NS
ROWS_PER_W = ROWS // NW
ELEMS_PER_W = ROWS_PER_W * C
IDX_MINOR = 128
IDX_MAJOR = ELEMS_PER_W // IDX_MINOR
OUT_ROWS = ELEMS // IDX_MINOR


@functools.partial(
    pl.kernel,
    out_type=jax.ShapeDtypeStruct((OUT_ROWS, IDX_MINOR), jnp.float32),
    mesh=plsc.VectorSubcoreMesh(core_axis_name="c", subcore_axis_name="s"),
    scratch_types=[
        pltpu.VMEM((ROWS_PER_W,), jnp.int32),
        pltpu.VMEM((IDX_MAJOR, IDX_MINOR), jnp.int32),
        pltpu.VMEM((IDX_MAJOR, IDX_MINOR), jnp.float32),
        pltpu.SemaphoreType.DMA,
    ],
)
def _sc_gather(feats_hbm, pids_hbm, out_hbm, pid_v, idx_v, rows_v, sem):
    wid = lax.axis_index("s") * NC + lax.axis_index("c")
    base_row = wid * ROWS_PER_W
    b = lax.shift_right_logical(wid, 3)
    base_b = b * CHW

    pltpu.sync_copy(pids_hbm.at[pl.ds(base_row, ROWS_PER_W)], pid_v)

    iota16 = lax.iota(jnp.int32, L)
    c_offs = [(iota16 + 16 * cc) * HW + base_b for cc in range(C // L)]

    for g in range(ROWS_PER_W // L):
        p = pid_v[pl.ds(g * L, L)]
        q = lax.shift_right_logical(p, 7)
        rem = lax.bitwise_and(p, 127)
        h = lax.shift_right_logical(q * 43691, 17)
        wq = q - 3 * h
        pid_vec = (
            lax.shift_right_logical(h, 3) * 3072
            + wq * 1024
            + lax.bitwise_and(h, 7) * 128
            + rem
        )
        for r in range(L):
            pid = pid_vec[r]
            row = g * L + r
            for cc in range(C // L):
                u = (C // L) * row + cc
                j, k = divmod(u, IDX_MINOR // L)
                idx_v[j, pl.ds(k * L, L)] = c_offs[cc] + pid

    def fire(j, carry):
        pltpu.async_copy(feats_hbm.at[idx_v.at[j]], rows_v.at[j], sem)
        return carry

    lax.fori_loop(0, IDX_MAJOR, fire, 0)
    pltpu.make_async_copy(
        out_hbm.at[pl.ds(wid * IDX_MAJOR, IDX_MAJOR)], rows_v, sem
    ).wait()
    pltpu.sync_copy(rows_v, out_hbm.at[pl.ds(wid * IDX_MAJOR, IDX_MAJOR)])


def _norm_body(x_ref, o_ref):
    x = x_ref[...]
    s = jnp.sum(x * x, axis=1, keepdims=True)
    o_ref[...] = x / (jnp.sqrt(s) + 1e-7)


def _normalize(x):
    return pl.pallas_call(
        _norm_body,
        out_shape=jax.ShapeDtypeStruct((ROWS, C), jnp.float32),
    )(x)


def kernel(feats, num_patches, patch_ids):
    del num_patches
    feats_flat = (
        feats.reshape(B, C, 48, 8, 3, 128)
        .transpose(0, 1, 2, 4, 3, 5)
        .reshape(-1)
    )
    pids_flat = patch_ids.reshape(-1)
    gathered = _sc_gather(feats_flat, pids_flat)
    normed = _normalize(gathered.reshape(ROWS, C))
    return normed.reshape(B, NUM_P, C), patch_ids


# speedup vs baseline: 25.0727x; 1.1198x over previous
import functools

import jax
import jax.numpy as jnp
from jax import lax
from jax.experimental import pallas as pl
from jax.experimental.pallas import tpu as pltpu
from jax.experimental.pallas import tpu_sc as plsc

B = 4
C = 192
HW = 384 * 384
CHW = C * HW
NUM_P = 256
ELEMS = B * NUM_P * C

_info = plsc.get_sparse_core_info()
NC, NS, L = _info.num_cores, _info.num_subcores, _info.num_lanes
NW = NC * NS
ELEMS_PER_W = ELEMS // NW
IDX_MINOR = 128
IDX_MAJOR = ELEMS_PER_W // IDX_MINOR
OUT_ROWS = ELEMS // IDX_MINOR
C8_PER_W = 3


@functools.partial(
    pl.kernel,
    out_type=jax.ShapeDtypeStruct((OUT_ROWS, IDX_MINOR), jnp.float32),
    mesh=plsc.VectorSubcoreMesh(core_axis_name="c", subcore_axis_name="s"),
    scratch_types=[
        pltpu.VMEM((NUM_P,), jnp.int32),
        pltpu.VMEM((IDX_MAJOR, IDX_MINOR), jnp.int32),
        pltpu.VMEM((IDX_MAJOR, IDX_MINOR), jnp.float32),
        pltpu.SemaphoreType.DMA,
    ],
)
def _sc_gather(feats_hbm, pids_hbm, out_hbm, fpid_v, idx_v, rows_v, sem):
    t = lax.axis_index("c") * NS + lax.axis_index("s")
    b = lax.shift_right_logical(t, 3)
    for p128 in range(NUM_P // IDX_MINOR):
        pltpu.sync_copy(
            pids_hbm.at[pl.ds((p128 * B + b) * IDX_MINOR, IDX_MINOR)],
            fpid_v.at[pl.ds(p128 * IDX_MINOR, IDX_MINOR)],
        )

    for k in range(NUM_P // L):
        p = fpid_v[pl.ds(k * L, L)]
        q = lax.shift_right_logical(p, 7)
        rem = lax.bitwise_and(p, 127)
        h = lax.shift_right_logical(q * 43691, 17)
        wq = q - 3 * h
        fpid_v[pl.ds(k * L, L)] = (
            lax.shift_right_logical(h, 3) * 3072
            + wq * 1024
            + lax.bitwise_and(h, 7) * 128
            + rem
        )

    base_b = b * CHW

    def row_fn(j, carry):
        c8r = lax.shift_right_logical(j, 4)
        p128 = lax.bitwise_and(lax.shift_right_logical(j, 3), 1)
        cm8 = lax.bitwise_and(j, 7)
        c = ((lax.bitwise_and(t, 7) * C8_PER_W + c8r) * 8) + cm8
        base = base_b + c * HW
        for kk in range(IDX_MINOR // L):
            idx_v[j, pl.ds(kk * L, L)] = (
                fpid_v[pl.ds(p128 * IDX_MINOR + kk * L, L)] + base
            )
        return carry

    lax.fori_loop(0, IDX_MAJOR, row_fn, 0)

    def fire(j, carry):
        pltpu.async_copy(feats_hbm.at[idx_v.at[j]], rows_v.at[j], sem)
        return carry

    lax.fori_loop(0, IDX_MAJOR, fire, 0)
    pltpu.make_async_copy(
        out_hbm.at[pl.ds(t * IDX_MAJOR, IDX_MAJOR)], rows_v, sem
    ).wait()
    pltpu.sync_copy(rows_v, out_hbm.at[pl.ds(t * IDX_MAJOR, IDX_MAJOR)])


def _norm_body(x_ref, o_ref):
    x = x_ref[...].reshape(B, C // 8, NUM_P // 128, 8, 128)
    s = jnp.sum(x * x, axis=(1, 3), keepdims=True)
    o = x / (jnp.sqrt(s) + 1e-7)
    o_ref[...] = o.reshape(OUT_ROWS, IDX_MINOR)


def _normalize(x):
    return pl.pallas_call(
        _norm_body,
        out_shape=jax.ShapeDtypeStruct((OUT_ROWS, IDX_MINOR), jnp.float32),
    )(x)


def kernel(feats, num_patches, patch_ids):
    del num_patches
    feats_flat = (
        feats.reshape(B, C, 48, 8, 3, 128)
        .transpose(0, 1, 2, 4, 3, 5)
        .reshape(-1)
    )
    pids_flat = (
        patch_ids.reshape(B, NUM_P // 128, 128)
        .transpose(1, 0, 2)
        .reshape(-1)
    )
    gathered = _sc_gather(feats_flat, pids_flat)
    normed = _normalize(gathered)
    out = (
        normed.reshape(B, C // 8, NUM_P // 128, 8, 128)
        .transpose(0, 2, 4, 1, 3)
        .reshape(B, NUM_P, C)
    )
    return out, patch_ids
